# Initial kernel scaffold; baseline (speedup 1.0000x reference)
import jax
import jax.numpy as jnp
from jax.experimental import pallas as pl


def kernel(concepts, embeddings, sigmas_table):
    raise NotImplementedError("write your pallas kernel here")


# speedup vs baseline: 1.9846x; 1.9846x over previous
import functools

import jax
import jax.numpy as jnp
import numpy as np
from jax import lax
from jax.experimental import pallas as pl
from jax.experimental.pallas import tpu as pltpu
from jax.experimental.pallas import tpu_sc as plsc

B, L, D = 1024, 200, 128
R = B * L

_KS0 = 0
_KS1 = 42
_KS2 = _KS0 ^ _KS1 ^ 0x1BD11BDA

_ROT_A = (13, 15, 26, 6)
_ROT_B = (17, 29, 16, 24)

_U_LO = float(np.nextafter(np.float32(-1.0), np.float32(0.0)))
_U_SCALE = 1.0 - _U_LO

_ERFINV_CEN = (2.81022636e-08, 3.43273939e-07, -3.5233877e-06,
               -4.39150654e-06, 0.00021858087, -0.00125372503,
               -0.00417768164, 0.246640727, 1.50140941)
_ERFINV_TAIL = (-0.000200214257, 0.000100950558, 0.00134934322,
                -0.00367342844, 0.00573950773, -0.0076224613,
                0.00943887047, 1.00167406, 2.83297682)

_SQRT2 = 1.4142135623730951


def _u32(v):
    return jnp.uint32(v & 0xFFFFFFFF)


def _noise_scale_body(sig_ref, out_ref, *, rows_per_block):
    i = pl.program_id(0)
    base = i * (rows_per_block * D)
    row = lax.broadcasted_iota(jnp.int32, (rows_per_block, D), 0)
    col = lax.broadcasted_iota(jnp.int32, (rows_per_block, D), 1)
    idx = (base + row * D + col).astype(jnp.uint32)

    x0 = jnp.full((rows_per_block, D), _u32(_KS0), jnp.uint32)
    x1 = idx + _u32(_KS1)

    def rounds(x0, x1, rots):
        for r in rots:
            x0 = x0 + x1
            x1 = (x1 << jnp.uint32(r)) | (x1 >> jnp.uint32(32 - r))
            x1 = x0 ^ x1
        return x0, x1

    x0, x1 = rounds(x0, x1, _ROT_A)
    x0, x1 = x0 + _u32(_KS1), x1 + _u32(_KS2 + 1)
    x0, x1 = rounds(x0, x1, _ROT_B)
    x0, x1 = x0 + _u32(_KS2), x1 + _u32(_KS0 + 2)
    x0, x1 = rounds(x0, x1, _ROT_A)
    x0, x1 = x0 + _u32(_KS0), x1 + _u32(_KS1 + 3)
    x0, x1 = rounds(x0, x1, _ROT_B)
    x0, x1 = x0 + _u32(_KS1), x1 + _u32(_KS2 + 4)
    x0, x1 = rounds(x0, x1, _ROT_A)
    x0, x1 = x0 + _u32(_KS2), x1 + _u32(_KS0 + 5)
    bits = x0 ^ x1

    fbits = (bits >> jnp.uint32(9)) | jnp.uint32(0x3F800000)
    f = lax.bitcast_convert_type(fbits, jnp.float32) - jnp.float32(1.0)
    u = jnp.maximum(jnp.float32(_U_LO),
                    f * jnp.float32(_U_SCALE) + jnp.float32(_U_LO))

    w = -jnp.log(jnp.float32(1.0) - u * u)
    cen = w < jnp.float32(5.0)
    z = jnp.where(cen, w - jnp.float32(2.5), jnp.sqrt(w) - jnp.float32(3.0))
    p = jnp.where(cen, jnp.float32(_ERFINV_CEN[0]), jnp.float32(_ERFINV_TAIL[0]))
    for cc, ct in zip(_ERFINV_CEN[1:], _ERFINV_TAIL[1:]):
        p = jnp.where(cen, jnp.float32(cc), jnp.float32(ct)) + p * z
    noise = jnp.float32(_SQRT2) * (p * u)

    out_ref[...] = noise * sig_ref[...]


def _noise_scale(sig, rows_per_block=1024):
    nblk = R // rows_per_block
    return pl.pallas_call(
        functools.partial(_noise_scale_body, rows_per_block=rows_per_block),
        grid=(nblk,),
        in_specs=[pl.BlockSpec((rows_per_block, 1), lambda i: (i, 0))],
        out_specs=pl.BlockSpec((rows_per_block, D), lambda i: (i, 0)),
        out_shape=jax.ShapeDtypeStruct((R, D), jnp.float32),
    )(sig)


def _sigma_gather(table_flat, idx_flat):
    n_table = table_flat.shape[0]
    info = plsc.get_sparse_core_info()
    nw = info.num_cores * info.num_subcores
    b_per_w = R // nw
    mesh = plsc.VectorSubcoreMesh(core_axis_name="c", subcore_axis_name="s")

    @functools.partial(
        pl.kernel, mesh=mesh,
        out_type=jax.ShapeDtypeStruct((R,), jnp.float32),
        compiler_params=pltpu.CompilerParams(needs_layout_passes=False),
        scratch_types=[
            pltpu.VMEM((n_table,), jnp.float32),
            pltpu.VMEM((b_per_w,), jnp.int32),
            pltpu.VMEM((b_per_w,), jnp.float32),
        ],
    )
    def gather_k(table_hbm, idx_hbm, out_hbm, table_v, idx_v, out_v):
        wid = lax.axis_index("s") * info.num_cores + lax.axis_index("c")
        base = wid * b_per_w
        pltpu.sync_copy(table_hbm, table_v)
        pltpu.sync_copy(idx_hbm.at[pl.ds(base, b_per_w)], idx_v)

        def body(j, carry):
            iv = idx_v[pl.ds(j * 16, 16)]
            out_v[pl.ds(j * 16, 16)] = plsc.load_gather(table_v, [iv])
            return carry

        lax.fori_loop(0, b_per_w // 16, body, 0)
        pltpu.sync_copy(out_v, out_hbm.at[pl.ds(base, b_per_w)])

    return gather_k(table_flat, idx_flat)


def kernel(concepts, embeddings, sigmas_table):
    del embeddings
    idx_flat = concepts.reshape(R).astype(jnp.int32)
    sig = _sigma_gather(sigmas_table.reshape(-1), idx_flat).reshape(R, 1)
    out = _noise_scale(sig)
    return out.reshape(B, L, D)
